# baseline (speedup 1.0000x reference)
import functools

import jax
import jax.numpy as jnp
from jax import lax
from jax.experimental import pallas as pl
from jax.experimental.pallas import tpu as pltpu
from jax.experimental.pallas import tpu_sc as plsc

_MAX_POS = 8192
_EMBED_DIM = 1024


def _make_sc_copy():
    info = plsc.get_sparse_core_info()
    nc, ns = info.num_cores, info.num_subcores
    nw = nc * ns
    rows_per_w = _MAX_POS // nw

    mesh = plsc.VectorSubcoreMesh(core_axis_name="c", subcore_axis_name="s")

    @functools.partial(
        pl.kernel,
        mesh=mesh,
        out_type=jax.ShapeDtypeStruct((_MAX_POS, _EMBED_DIM), jnp.float32),
    )
    def k(pe_hbm, out_hbm):
        wid = lax.axis_index("s") * nc + lax.axis_index("c")
        base = wid * rows_per_w
        pltpu.sync_copy(
            pe_hbm.at[pl.ds(base, rows_per_w)],
            out_hbm.at[pl.ds(base, rows_per_w)],
        )

    return k


_sc_copy = _make_sc_copy()


def kernel(x, pe):
    return _sc_copy(pe)[None]


# speedup vs baseline: 23.1582x; 23.1582x over previous
import functools

import jax
import jax.numpy as jnp
from jax import lax
from jax.experimental import pallas as pl
from jax.experimental.pallas import tpu as pltpu
from jax.experimental.pallas import tpu_sc as plsc

_MAX_POS = 8192
_EMBED_DIM = 1024
_CHUNK_ROWS = 32


def _make_sc_copy():
    info = plsc.get_sparse_core_info()
    nc, ns = info.num_cores, info.num_subcores
    nw = nc * ns
    rows_per_w = _MAX_POS // nw
    nchunk = rows_per_w // _CHUNK_ROWS

    mesh = plsc.VectorSubcoreMesh(core_axis_name="c", subcore_axis_name="s")

    @functools.partial(
        pl.kernel,
        mesh=mesh,
        out_type=jax.ShapeDtypeStruct((_MAX_POS, _EMBED_DIM), jnp.float32),
        scratch_types=[
            pltpu.VMEM((2, _CHUNK_ROWS, _EMBED_DIM), jnp.float32),
            pltpu.SemaphoreType.DMA,
            pltpu.SemaphoreType.DMA,
        ],
    )
    def k(pe_hbm, out_hbm, buf, in_sem, out_sem):
        wid = lax.axis_index("s") * nc + lax.axis_index("c")
        base = wid * rows_per_w

        def in_copy(i, slot):
            return pltpu.make_async_copy(
                pe_hbm.at[pl.ds(base + i * _CHUNK_ROWS, _CHUNK_ROWS)],
                buf.at[slot],
                in_sem,
            )

        def out_copy(i, slot):
            return pltpu.make_async_copy(
                buf.at[slot],
                out_hbm.at[pl.ds(base + i * _CHUNK_ROWS, _CHUNK_ROWS)],
                out_sem,
            )

        in_copy(0, 0).start()
        for i in range(nchunk):
            s = i % 2
            in_copy(i, s).wait()
            if i + 1 < nchunk:
                if i >= 1:
                    out_copy(i - 1, (i - 1) % 2).wait()
                in_copy(i + 1, (i + 1) % 2).start()
            out_copy(i, s).start()
        out_copy(nchunk - 2, (nchunk - 2) % 2).wait()
        out_copy(nchunk - 1, (nchunk - 1) % 2).wait()

    return k


_sc_copy = _make_sc_copy()


def kernel(x, pe):
    return _sc_copy(pe)[None]


# speedup vs baseline: 24.8927x; 1.0749x over previous
import functools

import jax
import jax.numpy as jnp
from jax import lax
from jax.experimental import pallas as pl
from jax.experimental.pallas import tpu as pltpu
from jax.experimental.pallas import tpu_sc as plsc

_MAX_POS = 8192
_EMBED_DIM = 1024
_CHUNK_ROWS = 32
_NBUF = 3


def _make_sc_copy():
    info = plsc.get_sparse_core_info()
    nc, ns = info.num_cores, info.num_subcores
    nw = nc * ns
    rows_per_w = _MAX_POS // nw
    nchunk = rows_per_w // _CHUNK_ROWS

    mesh = plsc.VectorSubcoreMesh(core_axis_name="c", subcore_axis_name="s")

    @functools.partial(
        pl.kernel,
        mesh=mesh,
        out_type=jax.ShapeDtypeStruct((_MAX_POS, _EMBED_DIM), jnp.float32),
        scratch_types=[
            pltpu.VMEM((_NBUF, _CHUNK_ROWS, _EMBED_DIM), jnp.float32),
            pltpu.SemaphoreType.DMA,
            pltpu.SemaphoreType.DMA,
        ],
    )
    def k(pe_hbm, out_hbm, buf, in_sem, out_sem):
        wid = lax.axis_index("s") * nc + lax.axis_index("c")
        base = wid * rows_per_w

        def in_copy(i, slot):
            return pltpu.make_async_copy(
                pe_hbm.at[pl.ds(base + i * _CHUNK_ROWS, _CHUNK_ROWS)],
                buf.at[slot],
                in_sem,
            )

        def out_copy(i, slot):
            return pltpu.make_async_copy(
                buf.at[slot],
                out_hbm.at[pl.ds(base + i * _CHUNK_ROWS, _CHUNK_ROWS)],
                out_sem,
            )

        for j in range(_NBUF - 1):
            in_copy(j, j).start()
        for i in range(nchunk):
            s = i % _NBUF
            in_copy(i, s).wait()
            nxt = i + _NBUF - 1
            if nxt < nchunk:
                if i >= 1:
                    out_copy(i - 1, (i - 1) % _NBUF).wait()
                in_copy(nxt, nxt % _NBUF).start()
            out_copy(i, s).start()
        for i in range(nchunk - _NBUF, nchunk):
            out_copy(i, i % _NBUF).wait()

    return k


_sc_copy = _make_sc_copy()


def kernel(x, pe):
    return _sc_copy(pe)[None]
